# baseline (device time: 17365 ns/iter reference)
import jax
import jax.numpy as jnp
from jax import lax
from jax.experimental import pallas as pl
from jax.experimental.pallas import tpu as pltpu

N_DEV = 4
B = 128


def kernel(x):
    m, n = x.shape
    nb = m // B

    def body(x_hbm, out_hbm, xbuf, ybuf, comm_ref,
             in_sems, out_sems, send_sems, recv_sems):
        my = lax.axis_index("i")

        in_copies = []
        for c in range(nb):
            cp = pltpu.make_async_copy(
                x_hbm.at[pl.ds(c * B, B), :],
                xbuf.at[pl.ds(c * B, B), :],
                in_sems.at[c],
            )
            cp.start()
            in_copies.append(cp)

        barrier_sem = pltpu.get_barrier_semaphore()
        for d in range(1, N_DEV):
            pl.semaphore_signal(
                barrier_sem, inc=1,
                device_id=((my + d) % N_DEV,),
                device_id_type=pl.DeviceIdType.MESH,
            )
        pl.semaphore_wait(barrier_sem, N_DEV - 1)

        rows = lax.broadcasted_iota(jnp.int32, (B, B), 0)
        cols = lax.broadcasted_iota(jnp.int32, (B, B), 1)
        L = jnp.where(rows >= cols, 1.0, 0.0).astype(jnp.bfloat16)

        p = jnp.zeros((1, n), jnp.float32)
        for c in range(nb):
            in_copies[c].wait()
            xb = xbuf[pl.ds(c * B, B), :]
            zb = jnp.dot(L, xb.astype(jnp.bfloat16),
                         preferred_element_type=jnp.float32)
            ybuf[pl.ds(c * B, B), :] = zb + p
            p = p + jnp.sum(xb, axis=0, keepdims=True)

        comm_ref[0, :] = p[0, :]
        rdmas = []
        for d in range(1, N_DEV):
            rdma = pltpu.make_async_remote_copy(
                src_ref=comm_ref.at[pl.ds(0, 1)],
                dst_ref=comm_ref.at[pl.ds(d, 1)],
                send_sem=send_sems.at[d - 1],
                recv_sem=recv_sems.at[d - 1],
                device_id=((my + d) % N_DEV,),
                device_id_type=pl.DeviceIdType.MESH,
            )
            rdma.start()
            rdmas.append(rdma)
        for rdma in rdmas:
            rdma.wait()

        hs = lax.broadcasted_iota(jnp.int32, (N_DEV, n), 0)
        mask = (hs >= 1) & (hs <= my)
        offset = jnp.sum(
            jnp.where(mask, comm_ref[:, :], 0.0), axis=0, keepdims=True
        )

        out_copies = []
        for c in range(nb):
            ybuf[pl.ds(c * B, B), :] = ybuf[pl.ds(c * B, B), :] + offset
            cp = pltpu.make_async_copy(
                ybuf.at[pl.ds(c * B, B), :],
                out_hbm.at[pl.ds(c * B, B), :],
                out_sems.at[c],
            )
            cp.start()
            out_copies.append(cp)
        for cp in out_copies:
            cp.wait()

    return pl.pallas_call(
        body,
        out_shape=jax.ShapeDtypeStruct((m, n), x.dtype),
        in_specs=[pl.BlockSpec(memory_space=pl.ANY)],
        out_specs=pl.BlockSpec(memory_space=pl.ANY),
        scratch_shapes=[
            pltpu.VMEM((m, n), jnp.float32),
            pltpu.VMEM((m, n), jnp.float32),
            pltpu.VMEM((N_DEV, n), jnp.float32),
            pltpu.SemaphoreType.DMA((nb,)),
            pltpu.SemaphoreType.DMA((nb,)),
            pltpu.SemaphoreType.DMA((N_DEV - 1,)),
            pltpu.SemaphoreType.DMA((N_DEV - 1,)),
        ],
        compiler_params=pltpu.CompilerParams(collective_id=0),
    )(x)
